# Initial kernel scaffold; baseline (speedup 1.0000x reference)
import functools

import jax
import jax.numpy as jnp
from jax.experimental import pallas as pl
from jax.experimental.pallas import tpu as pltpu

N = 10000
E = 320000
D = 128
H = 4
C = 64
HC = H * C
NPAD = 10240


def _qkvs_body(h_ref, w_ref, b_ref, o_ref):
    o_ref[...] = (
        jnp.dot(h_ref[...], w_ref[...], preferred_element_type=jnp.float32)
        + b_ref[...]
    )


def _qkvs(h, W4, b4, bm=2048):
    n, din = h.shape
    cols = W4.shape[1]
    return pl.pallas_call(
        _qkvs_body,
        grid=(n // bm,),
        in_specs=[
            pl.BlockSpec((bm, din), lambda i: (i, 0)),
            pl.BlockSpec((din, cols), lambda i: (0, 0)),
            pl.BlockSpec((1, cols), lambda i: (0, 0)),
        ],
        out_specs=pl.BlockSpec((bm, cols), lambda i: (i, 0)),
        out_shape=jax.ShapeDtypeStruct((n, cols), jnp.float32),
    )(h, W4, b4)


def _ln_body(a_ref, s_ref, g_ref, b_ref, o_ref, *, relu):
    x = a_ref[...] + s_ref[...]
    m = jnp.mean(x, axis=1, keepdims=True)
    xc = x - m
    v = jnp.mean(xc * xc, axis=1, keepdims=True)
    y = xc * jax.lax.rsqrt(v + 1e-5) * g_ref[...] + b_ref[...]
    if relu:
        y = jnp.maximum(y, 0.0)
    o_ref[...] = y


def _add_ln(agg, skip, g, b, relu, bm=2048):
    n, cols = agg.shape
    return pl.pallas_call(
        functools.partial(_ln_body, relu=relu),
        grid=(n // bm,),
        in_specs=[
            pl.BlockSpec((bm, cols), lambda i: (i, 0)),
            pl.BlockSpec((bm, cols), lambda i: (i, 0)),
            pl.BlockSpec((1, cols), lambda i: (0, 0)),
            pl.BlockSpec((1, cols), lambda i: (0, 0)),
        ],
        out_specs=pl.BlockSpec((bm, cols), lambda i: (i, 0)),
        out_shape=jax.ShapeDtypeStruct((n, cols), jnp.float32),
    )(agg, skip, g, b)


def _edge_phase(q, k, v, src, dst):
    alpha = (
        (q[dst].reshape(E, H, C) * k[src].reshape(E, H, C)).sum(-1)
        / jnp.sqrt(jnp.float32(C))
    )
    mx = jax.ops.segment_max(alpha, dst, num_segments=N)
    mx = jnp.where(jnp.isfinite(mx), mx, 0.0)
    e = jnp.exp(alpha - mx[dst])
    s = jax.ops.segment_sum(e, dst, num_segments=N)
    a = e / (s[dst] + 1e-16)
    agg = jax.ops.segment_sum(
        v[src].reshape(E, H, C) * a[:, :, None], dst, num_segments=N
    ).reshape(N, HC)
    return jnp.pad(agg, ((0, NPAD - N), (0, 0)))


def _layer(h, src, dst, Wq, bq, Wk, bk, Wv, bv, Ws, bs, g, b, relu):
    W4 = jnp.concatenate([Wq, Wk, Wv, Ws], axis=1)
    b4 = jnp.concatenate([bq, bk, bv, bs])[None, :]
    qkvs = _qkvs(h, W4, b4)
    q = qkvs[:, 0 * HC : 1 * HC]
    k = qkvs[:, 1 * HC : 2 * HC]
    v = qkvs[:, 2 * HC : 3 * HC]
    skip = qkvs[:, 3 * HC : 4 * HC]
    agg = _edge_phase(q, k, v, src, dst)
    return _add_ln(agg, skip, g[None, :], b[None, :], relu)


def kernel(x, edge_index, Wq1, bq1, Wk1, bk1, Wv1, bv1, Ws1, bs1, g1, b1,
           Wq2, bq2, Wk2, bk2, Wv2, bv2, Ws2, bs2, g2, b2):
    src = edge_index[0]
    dst = edge_index[1]
    h = jnp.pad(x, ((0, NPAD - N), (0, 0)))
    h = _layer(h, src, dst, Wq1, bq1, Wk1, bk1, Wv1, bv1, Ws1, bs1, g1, b1, True)
    h = _layer(h, src, dst, Wq2, bq2, Wk2, bk2, Wv2, bv2, Ws2, bs2, g2, b2, False)
    return h[:N]


# speedup vs baseline: 10.6214x; 10.6214x over previous
import functools

import jax
import jax.numpy as jnp
from jax import lax
from jax.experimental import pallas as pl
from jax.experimental.pallas import tpu as pltpu
from jax.experimental.pallas import tpu_sc as plsc

N = 10000
E = 320000
D = 128
H = 4
C = 64
HC = H * C
NPAD = 10240

NSC = 2
NT = 16
HW = HC // NSC
HWE = HW + 16
BE = 64
EPAD = 320512
ECP = EPAD // NT
NSTEP = ECP // BE
NROW = NPAD // NT
WBCH = 40


def _qkvs_body(h_ref, w_ref, b_ref, o_ref):
    o_ref[...] = (
        jnp.dot(h_ref[...], w_ref[...], preferred_element_type=jnp.float32)
        + b_ref[...]
    )


def _qkvs(h, W4, b4, bm=2048):
    n, din = h.shape
    cols = W4.shape[1]
    return pl.pallas_call(
        _qkvs_body,
        grid=(n // bm,),
        in_specs=[
            pl.BlockSpec((bm, din), lambda i: (i, 0)),
            pl.BlockSpec((din, cols), lambda i: (0, 0)),
            pl.BlockSpec((1, cols), lambda i: (0, 0)),
        ],
        out_specs=pl.BlockSpec((bm, cols), lambda i: (i, 0)),
        out_shape=jax.ShapeDtypeStruct((n, cols), jnp.float32),
    )(h, W4, b4)


def _ln_body(a_ref, s_ref, g_ref, b_ref, o_ref, *, relu):
    x = a_ref[...] + s_ref[...]
    m = jnp.mean(x, axis=1, keepdims=True)
    xc = x - m
    v = jnp.mean(xc * xc, axis=1, keepdims=True)
    y = xc * jax.lax.rsqrt(v + 1e-5) * g_ref[...] + b_ref[...]
    if relu:
        y = jnp.maximum(y, 0.0)
    o_ref[...] = y


def _add_ln(agg, skip, g, b, relu, bm=2048):
    n, cols = agg.shape
    return pl.pallas_call(
        functools.partial(_ln_body, relu=relu),
        grid=(n // bm,),
        in_specs=[
            pl.BlockSpec((bm, cols), lambda i: (i, 0)),
            pl.BlockSpec((bm, cols), lambda i: (i, 0)),
            pl.BlockSpec((1, cols), lambda i: (0, 0)),
            pl.BlockSpec((1, cols), lambda i: (0, 0)),
        ],
        out_specs=pl.BlockSpec((bm, cols), lambda i: (i, 0)),
        out_shape=jax.ShapeDtypeStruct((n, cols), jnp.float32),
    )(agg, skip, g, b)


def _edge_body(q2, k2, v2, srcd, dstd, agg_out,
               src_v, dst_v, qidx_v, qrows, krows, vrows, wbuf, aexp,
               agg_sh, sem):
    core = lax.axis_index("c")
    wid = lax.axis_index("s")
    coff = core * NPAD
    zero16 = jnp.zeros((16,), jnp.float32)
    lane = lax.iota(jnp.int32, 16)

    def zrow(i, _):
        for j in range(HWE // 16):
            wbuf[i, pl.ds(j * 16, 16)] = zero16
        return 0
    lax.fori_loop(0, WBCH, zrow, 0)
    for t in range(NROW // WBCH):
        pltpu.sync_copy(wbuf.at[pl.ds(0, WBCH)],
                        agg_sh.at[pl.ds(wid * NROW + t * WBCH, WBCH)])
    plsc.subcore_barrier()

    ebase0 = wid * ECP

    def step(si, _):
        base = ebase0 + si * BE
        pltpu.sync_copy(dstd.at[pl.ds(base, BE)], dst_v)
        pltpu.sync_copy(srcd.at[pl.ds(base, BE)], src_v)
        for j in range(BE // 16):
            sl = pl.ds(j * 16, 16)
            qidx_v[sl] = dst_v[sl] + coff
            src_v[sl] = src_v[sl] + coff
        cq = pltpu.async_copy(q2.at[qidx_v], qrows, sem)
        cq.wait()
        ck = pltpu.async_copy(k2.at[src_v], krows, sem)
        ck.wait()
        cv = pltpu.async_copy(v2.at[src_v], vrows, sem)
        cv.wait()

        def galpha(g, _):
            a0v = zero16
            a1v = zero16
            for ln in range(16):
                e = g * 16 + ln
                acc0 = zero16
                acc1 = zero16
                for j in range(4):
                    acc0 = acc0 + qrows[e, pl.ds(j * 16, 16)] * krows[e, pl.ds(j * 16, 16)]
                for j in range(4, 8):
                    acc1 = acc1 + qrows[e, pl.ds(j * 16, 16)] * krows[e, pl.ds(j * 16, 16)]
                m = lane == ln
                a0v = jnp.where(m, jnp.full((16,), jnp.sum(acc0)), a0v)
                a1v = jnp.where(m, jnp.full((16,), jnp.sum(acc1)), a1v)
            aexp[0, pl.ds(g * 16, 16)] = jnp.exp(a0v * 0.125)
            aexp[1, pl.ds(g * 16, 16)] = jnp.exp(a1v * 0.125)
            return 0
        lax.fori_loop(0, BE // 16, galpha, 0)

        def gscale(g, _):
            w0v = aexp[0, pl.ds(g * 16, 16)]
            w1v = aexp[1, pl.ds(g * 16, 16)]
            for ln in range(16):
                e = g * 16 + ln
                w0 = jnp.full((16,), w0v[ln])
                w1 = jnp.full((16,), w1v[ln])
                for r in range(4):
                    sl = pl.ds(r * 16, 16)
                    wbuf[e, sl] = vrows[e, sl] * w0
                for r in range(4, 8):
                    sl = pl.ds(r * 16, 16)
                    wbuf[e, sl] = vrows[e, sl] * w1
                ecol = jnp.where(lane == 0, w0, jnp.where(lane == 1, w1, zero16))
                wbuf[e, pl.ds(HW, 16)] = ecol
            return 0
        lax.fori_loop(0, BE // 16, gscale, 0)

        pltpu.sync_copy(wbuf, agg_sh.at[dst_v], add=True)
        return 0

    lax.fori_loop(0, NSTEP, step, 0)
    plsc.subcore_barrier()

    rbase = wid * NROW
    for t in range(NROW // WBCH):
        rb = rbase + t * WBCH
        pltpu.sync_copy(agg_sh.at[pl.ds(rb, WBCH)], wbuf.at[pl.ds(0, WBCH)])

        def norm(i, _):
            sv = wbuf[i, pl.ds(HW, 16)]
            winv = jnp.where(sv > 0.0, 1.0 / jnp.where(sv > 0.0, sv, 1.0), 0.0)
            w0 = jnp.full((16,), winv[0])
            w1 = jnp.full((16,), winv[1])
            for r in range(4):
                sl = pl.ds(r * 16, 16)
                qrows[i, sl] = wbuf[i, sl] * w0
            for r in range(4, 8):
                sl = pl.ds(r * 16, 16)
                qrows[i, sl] = wbuf[i, sl] * w1
            return 0
        lax.fori_loop(0, WBCH, norm, 0)
        pltpu.sync_copy(qrows.at[pl.ds(0, WBCH)],
                        agg_out.at[core, pl.ds(rb, WBCH)])


_edge_kernel = functools.partial(
    pl.kernel,
    out_type=jax.ShapeDtypeStruct((NSC, NPAD, HW), jnp.float32),
    mesh=plsc.VectorSubcoreMesh(core_axis_name="c", subcore_axis_name="s"),
    compiler_params=pltpu.CompilerParams(
        needs_layout_passes=False, use_tc_tiling_on_sc=False),
    scratch_types=[
        pltpu.VMEM((BE,), jnp.int32),
        pltpu.VMEM((BE,), jnp.int32),
        pltpu.VMEM((BE,), jnp.int32),
        pltpu.VMEM((BE, HW), jnp.float32),
        pltpu.VMEM((BE, HW), jnp.float32),
        pltpu.VMEM((BE, HW), jnp.float32),
        pltpu.VMEM((BE, HWE), jnp.float32),
        pltpu.VMEM((2, BE), jnp.float32),
        pltpu.VMEM_SHARED((NPAD, HWE), jnp.float32),
        pltpu.SemaphoreType.DMA,
    ],
)(_edge_body)


def _edge_phase(qkvs, srcp, dstp):
    q = qkvs[:, 0 * HC:1 * HC]
    k = qkvs[:, 1 * HC:2 * HC]
    v = qkvs[:, 2 * HC:3 * HC]

    def stack(a):
        return a.reshape(NPAD, NSC, HW).transpose(1, 0, 2).reshape(NSC * NPAD, HW)

    agg2 = _edge_kernel(stack(q), stack(k), stack(v), srcp, dstp)
    return agg2.transpose(1, 0, 2).reshape(NPAD, HC)


def _layer(h, srcp, dstp, Wq, bq, Wk, bk, Wv, bv, Ws, bs, g, b, relu):
    W4 = jnp.concatenate([Wq, Wk, Wv, Ws], axis=1)
    b4 = jnp.concatenate([bq, bk, bv, bs])[None, :]
    qkvs = _qkvs(h, W4, b4)
    skip = qkvs[:, 3 * HC:4 * HC]
    agg = _edge_phase(qkvs, srcp, dstp)
    return _add_ln(agg, skip, g[None, :], b[None, :], relu)


def kernel(x, edge_index, Wq1, bq1, Wk1, bk1, Wv1, bv1, Ws1, bs1, g1, b1,
           Wq2, bq2, Wk2, bk2, Wv2, bv2, Ws2, bs2, g2, b2):
    src = edge_index[0]
    dst = edge_index[1]
    srcp = jnp.concatenate([src, jnp.zeros((EPAD - E,), jnp.int32)])
    dstp = jnp.concatenate([dst, jnp.full((EPAD - E,), N, jnp.int32)])
    h = jnp.pad(x, ((0, NPAD - N), (0, 0)))
    h = _layer(h, srcp, dstp, Wq1, bq1, Wk1, bk1, Wv1, bv1, Ws1, bs1, g1, b1, True)
    h = _layer(h, srcp, dstp, Wq2, bq2, Wk2, bk2, Wv2, bv2, Ws2, bs2, g2, b2, False)
    return h[:N]
